# baseline (speedup 1.0000x reference)
import functools

import jax
import jax.numpy as jnp
import numpy as np
from jax import lax
from jax.experimental import pallas as pl
from jax.experimental.pallas import tpu as pltpu
from jax.experimental.pallas import tpu_sc as plsc

N = 10000
F = 128
E = 320000
NEG = 0.2

NW = 32
CB = 128
TCH = 2560
CH0 = 80
CH1 = 80
CHMAX = max(CH0, CH1)
EP = TCH * CB
NROWS = 10240
RPT = NROWS // 16

_mesh = plsc.VectorSubcoreMesh(core_axis_name="c", subcore_axis_name="s")


@functools.partial(
    pl.kernel,
    out_type=jax.ShapeDtypeStruct((2, RPT, 16), jnp.float32),
    mesh=_mesh,
    scratch_types=[
        pltpu.VMEM((CHMAX, CB), jnp.int32),
        pltpu.VMEM((NROWS,), jnp.float32),
        pltpu.VMEM((RPT, 16), jnp.float32),
        pltpu.VMEM((5, CB), jnp.int32),
        pltpu.VMEM_SHARED((RPT, 16), jnp.float32),
    ],
    compiler_params=pltpu.CompilerParams(
        needs_layout_passes=False, use_tc_tiling_on_sc=False
    ),
)
def _deg(c_hbm, out_hbm, idx_c, dv, dv2, ident, sdeg):
    cid = lax.axis_index("c")
    sid = lax.axis_index("s")
    start = jnp.where(cid == 0, sid * CH0, 16 * CH0 + sid * CH1)
    mych = jnp.where(cid == 0, CH0, CH1)
    pltpu.sync_copy(c_hbm.at[pl.ds(start, CHMAX)], idx_c)

    zero = jnp.zeros((16,), jnp.float32)

    @pl.loop(0, RPT)
    def _(i):
        dv[pl.ds(i * 16, 16)] = zero
        dv2[i, :] = zero

    lanes = lax.iota(jnp.int32, 16)
    for a in range(5):
        for b in range(8):
            ident[a, pl.ds(b * 16, 16)] = lanes + (a * CB + b * 16)

    @pl.when(sid == 0)
    def _():
        pltpu.sync_copy(dv2, sdeg)
    plsc.subcore_barrier()

    ones = jnp.ones((16,), jnp.float32)

    @pl.loop(0, mych)
    def _(g):
        for j in range(8):
            cval = idx_c[g, pl.ds(j * 16, 16)]
            plsc.addupdate_scatter(dv, [cval], ones)

    @pl.loop(0, RPT)
    def _(i):
        dv2[i, :] = dv[pl.ds(i * 16, 16)]

    for j in range(5):
        pltpu.sync_copy(dv2.at[pl.ds(j * CB, CB)], sdeg.at[ident.at[j]], add=True)
    plsc.subcore_barrier()

    @pl.when(sid == 0)
    def _():
        pltpu.sync_copy(sdeg, out_hbm.at[cid])


FH = F // 2

_ph = np.empty((FH,), np.int32)
for _g in range(FH // 32):
    for _k in range(16):
        _ph[32 * _g + 2 * _k] = 32 * _g + _k
        _ph[32 * _g + 2 * _k + 1] = 32 * _g + 16 + _k
_PMAT = np.zeros((FH, FH), np.float32)
for _m in range(FH):
    _PMAT[_ph[_m], _m] = 1.0


@functools.partial(
    pl.kernel,
    out_type=jax.ShapeDtypeStruct((2, NROWS, F), jnp.float32),
    mesh=_mesh,
    scratch_types=[
        pltpu.VMEM((CHMAX, CB), jnp.int32),
        pltpu.VMEM((CHMAX, CB), jnp.int32),
        pltpu.VMEM((CB, FH), jnp.bfloat16),
        pltpu.VMEM((CB, FH), jnp.bfloat16),
        pltpu.VMEM((CB, FH), jnp.bfloat16),
        pltpu.VMEM((CB, FH), jnp.bfloat16),
        pltpu.VMEM((CB, FH), jnp.float32),
        pltpu.VMEM((CB, FH), jnp.float32),
        pltpu.VMEM_SHARED((NROWS, FH), jnp.float32),
        pltpu.SemaphoreType.DMA,
        pltpu.SemaphoreType.DMA,
        pltpu.SemaphoreType.DMA,
        pltpu.SemaphoreType.DMA,
    ],
    compiler_params=pltpu.CompilerParams(
        needs_layout_passes=False, use_tc_tiling_on_sc=False
    ),
)
def _agg(ya_hbm, yb_hbm, r_hbm, c_hbm, out_hbm,
         idx_r, idx_c, rows0, rows1, rows2, rows3, stage, zbuf, acc,
         sem0, sem1, sem2, sem3):
    cid = lax.axis_index("c")
    sid = lax.axis_index("s")
    start = jnp.where(cid == 0, sid * CH0, 16 * CH0 + sid * CH1)
    mych = jnp.where(cid == 0, CH0, CH1)

    pltpu.sync_copy(r_hbm.at[pl.ds(start, CHMAX)], idx_r)
    pltpu.sync_copy(c_hbm.at[pl.ds(start, CHMAX)], idx_c)

    zero = jnp.zeros((16,), jnp.float32)

    @pl.loop(0, CB)
    def _(i):
        for j in range(FH // 16):
            zbuf[i, pl.ds(j * 16, 16)] = zero

    base = sid * RPT
    NBUF = 4
    bufs = (rows0, rows1, rows2, rows3)
    sems = (sem0, sem1, sem2, sem3)

    for p, y_hbm in enumerate((ya_hbm, yb_hbm)):
        for k in range(RPT // CB):
            pltpu.sync_copy(zbuf, acc.at[pl.ds(base + k * CB, CB)])
        plsc.subcore_barrier()

        for b in range(NBUF):
            pltpu.async_copy(y_hbm.at[idx_r.at[b]], bufs[b], sems[b])

        @pl.loop(0, mych, step=NBUF)
        def _(g0):
            for b in range(NBUF):
                g = g0 + b
                cur, csem = bufs[b], sems[b]
                pltpu.make_async_copy(y_hbm.at[idx_r.at[g]], cur, csem).wait()

                @pl.loop(0, CB, unroll=8)
                def _(i):
                    for q in range(FH // 32):
                        v = cur[i, pl.ds(q * 32, 32)]
                        lo, hi = plsc.unpack(
                            v,
                            format=plsc.PackFormat.INTERLEAVED,
                            preferred_element_type=jnp.float32,
                        )
                        stage[i, pl.ds(q * 32, 16)] = lo
                        stage[i, pl.ds(q * 32 + 16, 16)] = hi

                pltpu.sync_copy(stage, acc.at[idx_c.at[g]], add=True)

                @pl.when(g + NBUF < mych)
                def _():
                    pltpu.async_copy(y_hbm.at[idx_r.at[g + NBUF]], cur, csem)

        plsc.subcore_barrier()
        pltpu.sync_copy(
            acc.at[pl.ds(base, RPT)],
            out_hbm.at[cid, pl.ds(base, RPT), pl.ds(p * FH, FH)],
        )


BM = 1024


def _dinv_of(deg_ref):
    deg = deg_ref[0] + deg_ref[1] + 1.0
    return lax.rsqrt(deg)


def _mm1_body(deg_ref, x_ref, w_ref, p_ref, ya_ref, yb_ref, z_ref):
    xw = jnp.dot(x_ref[...], w_ref[...], preferred_element_type=jnp.float32)
    dinv = _dinv_of(deg_ref)
    y = xw * dinv[:, None]
    pm = p_ref[...]
    ya_ref[...] = jnp.dot(y[:, :FH], pm,
                          preferred_element_type=jnp.float32).astype(jnp.bfloat16)
    yb_ref[...] = jnp.dot(y[:, FH:], pm,
                          preferred_element_type=jnp.float32).astype(jnp.bfloat16)
    z_ref[...] = xw * (dinv * dinv)[:, None]


def _mm1(deg2, xp, W1, Pm):
    return pl.pallas_call(
        _mm1_body,
        grid=(NROWS // BM,),
        in_specs=[
            pl.BlockSpec((2, BM), lambda i: (0, i)),
            pl.BlockSpec((BM, F), lambda i: (i, 0)),
            pl.BlockSpec((F, F), lambda i: (0, 0)),
            pl.BlockSpec((FH, FH), lambda i: (0, 0)),
        ],
        out_specs=[
            pl.BlockSpec((BM, FH), lambda i: (i, 0)),
            pl.BlockSpec((BM, FH), lambda i: (i, 0)),
            pl.BlockSpec((BM, F), lambda i: (i, 0)),
        ],
        out_shape=[
            jax.ShapeDtypeStruct((NROWS, FH), jnp.bfloat16),
            jax.ShapeDtypeStruct((NROWS, FH), jnp.bfloat16),
            jax.ShapeDtypeStruct((NROWS, F), jnp.float32),
        ],
    )(deg2, xp, W1, Pm)


def _leaky(v):
    return jnp.where(v >= 0, v, NEG * v)


def _mm2_body(a_ref, z_ref, deg_ref, b_ref, w_ref, p_ref,
              h_ref, ya_ref, yb_ref, z2_ref):
    dinv = _dinv_of(deg_ref)
    a = a_ref[0] + a_ref[1]
    h = _leaky(a * dinv[:, None] + z_ref[...] + b_ref[...][None, :])
    h_ref[...] = h
    xw = jnp.dot(h, w_ref[...], preferred_element_type=jnp.float32)
    y = xw * dinv[:, None]
    pm = p_ref[...]
    ya_ref[...] = jnp.dot(y[:, :FH], pm,
                          preferred_element_type=jnp.float32).astype(jnp.bfloat16)
    yb_ref[...] = jnp.dot(y[:, FH:], pm,
                          preferred_element_type=jnp.float32).astype(jnp.bfloat16)
    z2_ref[...] = xw * (dinv * dinv)[:, None]


def _mm2(a1, z1, deg2, b1, W2, Pm):
    return pl.pallas_call(
        _mm2_body,
        grid=(NROWS // BM,),
        in_specs=[
            pl.BlockSpec((2, BM, F), lambda i: (0, i, 0)),
            pl.BlockSpec((BM, F), lambda i: (i, 0)),
            pl.BlockSpec((2, BM), lambda i: (0, i)),
            pl.BlockSpec((F,), lambda i: (0,)),
            pl.BlockSpec((F, F), lambda i: (0, 0)),
            pl.BlockSpec((FH, FH), lambda i: (0, 0)),
        ],
        out_specs=[
            pl.BlockSpec((BM, F), lambda i: (i, 0)),
            pl.BlockSpec((BM, FH), lambda i: (i, 0)),
            pl.BlockSpec((BM, FH), lambda i: (i, 0)),
            pl.BlockSpec((BM, F), lambda i: (i, 0)),
        ],
        out_shape=[
            jax.ShapeDtypeStruct((NROWS, F), jnp.float32),
            jax.ShapeDtypeStruct((NROWS, FH), jnp.bfloat16),
            jax.ShapeDtypeStruct((NROWS, FH), jnp.bfloat16),
            jax.ShapeDtypeStruct((NROWS, F), jnp.float32),
        ],
    )(a1, z1, deg2, b1, W2, Pm)


def _mm3_body(a_ref, z_ref, deg_ref, b_ref, h1_ref, wt_ref, wb_ref, bh_ref, o_ref):
    dinv = _dinv_of(deg_ref)
    a = a_ref[0] + a_ref[1]
    h2 = _leaky(a * dinv[:, None] + z_ref[...] + b_ref[...][None, :])
    o_ref[...] = (
        jnp.dot(h1_ref[...], wt_ref[...], preferred_element_type=jnp.float32)
        + jnp.dot(h2, wb_ref[...], preferred_element_type=jnp.float32)
        + bh_ref[...][None, :]
    )


def _mm3(a2, z2, deg2, b2, h1, Wt, Wb, bh):
    return pl.pallas_call(
        _mm3_body,
        grid=(NROWS // BM,),
        in_specs=[
            pl.BlockSpec((2, BM, F), lambda i: (0, i, 0)),
            pl.BlockSpec((BM, F), lambda i: (i, 0)),
            pl.BlockSpec((2, BM), lambda i: (0, i)),
            pl.BlockSpec((F,), lambda i: (0,)),
            pl.BlockSpec((BM, F), lambda i: (i, 0)),
            pl.BlockSpec((F, F), lambda i: (0, 0)),
            pl.BlockSpec((F, F), lambda i: (0, 0)),
            pl.BlockSpec((F,), lambda i: (0,)),
        ],
        out_specs=pl.BlockSpec((BM, F), lambda i: (i, 0)),
        out_shape=jax.ShapeDtypeStruct((N, F), jnp.float32),
    )(a2, z2, deg2, b2, h1, Wt, Wb, bh)


def kernel(x, edge_index, W1, b1, W2, b2, Wh, bh):
    row = edge_index[0]
    col = edge_index[1]
    pad = (TCH + CHMAX) * CB - E
    dummy = (jnp.arange(pad, dtype=jnp.int32) % (NROWS - N)) + N
    r3 = jnp.concatenate([row, jnp.zeros((pad,), jnp.int32)]).reshape(TCH + CHMAX, CB)
    c3 = jnp.concatenate([col, dummy]).reshape(TCH + CHMAX, CB)
    xp = jnp.pad(x, ((0, NROWS - N), (0, 0)))

    degp = _deg(c3)
    deg2 = degp.reshape(2, NROWS)
    Pm = jnp.asarray(_PMAT)

    y1a, y1b, z1 = _mm1(deg2, xp, W1, Pm)
    a1 = _agg(y1a, y1b, r3, c3)
    h1, y2a, y2b, z2 = _mm2(a1, z1, deg2, b1, W2, Pm)
    a2 = _agg(y2a, y2b, r3, c3)
    return _mm3(a2, z2, deg2, b2, h1, Wh[:F], Wh[F:], bh)


# speedup vs baseline: 1.0728x; 1.0728x over previous
import functools

import jax
import jax.numpy as jnp
import numpy as np
from jax import lax
from jax.experimental import pallas as pl
from jax.experimental.pallas import tpu as pltpu
from jax.experimental.pallas import tpu_sc as plsc

N = 10000
F = 128
E = 320000
NEG = 0.2

NW = 32
CB = 128
TCH = 2560
CH0 = 80
CH1 = 80
CHMAX = max(CH0, CH1)
EP = TCH * CB
NROWS = 10240
RPT = NROWS // 16

_mesh = plsc.VectorSubcoreMesh(core_axis_name="c", subcore_axis_name="s")


@functools.partial(
    pl.kernel,
    out_type=jax.ShapeDtypeStruct((2, RPT, 16), jnp.float32),
    mesh=_mesh,
    scratch_types=[
        pltpu.VMEM((CHMAX, CB), jnp.int32),
        pltpu.VMEM((NROWS,), jnp.float32),
        pltpu.VMEM((RPT, 16), jnp.float32),
        pltpu.VMEM((5, CB), jnp.int32),
        pltpu.VMEM_SHARED((RPT, 16), jnp.float32),
    ],
    compiler_params=pltpu.CompilerParams(
        needs_layout_passes=False, use_tc_tiling_on_sc=False
    ),
)
def _deg(c_hbm, out_hbm, idx_c, dv, dv2, ident, sdeg):
    cid = lax.axis_index("c")
    sid = lax.axis_index("s")
    start = jnp.where(cid == 0, sid * CH0, 16 * CH0 + sid * CH1)
    mych = jnp.where(cid == 0, CH0, CH1)
    pltpu.sync_copy(c_hbm.at[pl.ds(start, CHMAX)], idx_c)

    zero = jnp.zeros((16,), jnp.float32)

    @pl.loop(0, RPT)
    def _(i):
        dv[pl.ds(i * 16, 16)] = zero
        dv2[i, :] = zero

    lanes = lax.iota(jnp.int32, 16)
    for a in range(5):
        for b in range(8):
            ident[a, pl.ds(b * 16, 16)] = lanes + (a * CB + b * 16)

    @pl.when(sid == 0)
    def _():
        pltpu.sync_copy(dv2, sdeg)
    plsc.subcore_barrier()

    ones = jnp.ones((16,), jnp.float32)

    @pl.loop(0, mych)
    def _(g):
        for j in range(8):
            cval = idx_c[g, pl.ds(j * 16, 16)]
            plsc.addupdate_scatter(dv, [cval], ones)

    @pl.loop(0, RPT)
    def _(i):
        dv2[i, :] = dv[pl.ds(i * 16, 16)]

    for j in range(5):
        pltpu.sync_copy(dv2.at[pl.ds(j * CB, CB)], sdeg.at[ident.at[j]], add=True)
    plsc.subcore_barrier()

    @pl.when(sid == 0)
    def _():
        pltpu.sync_copy(sdeg, out_hbm.at[cid])


FH = F // 2

_ph = np.empty((FH,), np.int32)
for _g in range(FH // 32):
    for _k in range(16):
        _ph[32 * _g + 2 * _k] = 32 * _g + _k
        _ph[32 * _g + 2 * _k + 1] = 32 * _g + 16 + _k
_PMAT = np.zeros((FH, FH), np.float32)
for _m in range(FH):
    _PMAT[_ph[_m], _m] = 1.0


@functools.partial(
    pl.kernel,
    out_type=jax.ShapeDtypeStruct((2, NROWS, F), jnp.float32),
    mesh=_mesh,
    scratch_types=[
        pltpu.VMEM((CHMAX, CB), jnp.int32),
        pltpu.VMEM((CHMAX, CB), jnp.int32),
        pltpu.VMEM((CB, FH), jnp.bfloat16),
        pltpu.VMEM((CB, FH), jnp.bfloat16),
        pltpu.VMEM((CB, FH), jnp.bfloat16),
        pltpu.VMEM((CB, FH), jnp.bfloat16),
        pltpu.VMEM((CB, FH), jnp.float32),
        pltpu.VMEM((CB, FH), jnp.float32),
        pltpu.VMEM((CB, FH), jnp.float32),
        pltpu.VMEM_SHARED((NROWS, FH), jnp.float32),
        pltpu.SemaphoreType.DMA,
        pltpu.SemaphoreType.DMA,
        pltpu.SemaphoreType.DMA,
        pltpu.SemaphoreType.DMA,
        pltpu.SemaphoreType.DMA,
        pltpu.SemaphoreType.DMA,
    ],
    compiler_params=pltpu.CompilerParams(
        needs_layout_passes=False, use_tc_tiling_on_sc=False
    ),
)
def _agg(ya_hbm, yb_hbm, r_hbm, c_hbm, out_hbm,
         idx_r, idx_c, rows0, rows1, rows2, rows3, stage0, stage1, zbuf, acc,
         sem0, sem1, sem2, sem3, ssem0, ssem1):
    cid = lax.axis_index("c")
    sid = lax.axis_index("s")
    start = jnp.where(cid == 0, sid * CH0, 16 * CH0 + sid * CH1)
    mych = jnp.where(cid == 0, CH0, CH1)

    pltpu.sync_copy(r_hbm.at[pl.ds(start, CHMAX)], idx_r)
    pltpu.sync_copy(c_hbm.at[pl.ds(start, CHMAX)], idx_c)

    zero = jnp.zeros((16,), jnp.float32)

    @pl.loop(0, CB)
    def _(i):
        for j in range(FH // 16):
            zbuf[i, pl.ds(j * 16, 16)] = zero

    base = sid * RPT
    NBUF = 4
    bufs = (rows0, rows1, rows2, rows3)
    sems = (sem0, sem1, sem2, sem3)

    for p, y_hbm in enumerate((ya_hbm, yb_hbm)):
        for k in range(RPT // CB):
            pltpu.sync_copy(zbuf, acc.at[pl.ds(base + k * CB, CB)])
        plsc.subcore_barrier()

        for b in range(NBUF):
            pltpu.async_copy(y_hbm.at[idx_r.at[b]], bufs[b], sems[b])

        stages = (stage0, stage1)
        ssems = (ssem0, ssem1)

        @pl.loop(0, mych, step=NBUF)
        def _(g0):
            for b in range(NBUF):
                g = g0 + b
                cur, csem = bufs[b], sems[b]
                stg, ssem = stages[b % 2], ssems[b % 2]
                pltpu.make_async_copy(y_hbm.at[idx_r.at[g]], cur, csem).wait()

                @pl.when(g >= 2)
                def _():
                    pltpu.make_async_copy(
                        stg, acc.at[idx_c.at[g - 2]], ssem
                    ).wait()

                @pl.loop(0, CB, unroll=4)
                def _(i):
                    for q in range(FH // 32):
                        v = cur[i, pl.ds(q * 32, 32)]
                        lo, hi = plsc.unpack(
                            v,
                            format=plsc.PackFormat.INTERLEAVED,
                            preferred_element_type=jnp.float32,
                        )
                        stg[i, pl.ds(q * 32, 16)] = lo
                        stg[i, pl.ds(q * 32 + 16, 16)] = hi

                pltpu.async_copy(stg, acc.at[idx_c.at[g]], ssem, add=True)

                @pl.when(g + NBUF < mych)
                def _():
                    pltpu.async_copy(y_hbm.at[idx_r.at[g + NBUF]], cur, csem)

        pltpu.make_async_copy(stage0, acc.at[idx_c.at[mych - 2]], ssem0).wait()
        pltpu.make_async_copy(stage1, acc.at[idx_c.at[mych - 1]], ssem1).wait()

        plsc.subcore_barrier()
        pltpu.sync_copy(
            acc.at[pl.ds(base, RPT)],
            out_hbm.at[cid, pl.ds(base, RPT), pl.ds(p * FH, FH)],
        )


BM = 1024


def _dinv_of(deg_ref):
    deg = deg_ref[0] + deg_ref[1] + 1.0
    return lax.rsqrt(deg)


def _mm1_body(deg_ref, x_ref, w_ref, p_ref, ya_ref, yb_ref, z_ref):
    xw = jnp.dot(x_ref[...], w_ref[...], preferred_element_type=jnp.float32)
    dinv = _dinv_of(deg_ref)
    y = xw * dinv[:, None]
    pm = p_ref[...]
    ya_ref[...] = jnp.dot(y[:, :FH], pm,
                          preferred_element_type=jnp.float32).astype(jnp.bfloat16)
    yb_ref[...] = jnp.dot(y[:, FH:], pm,
                          preferred_element_type=jnp.float32).astype(jnp.bfloat16)
    z_ref[...] = xw * (dinv * dinv)[:, None]


def _mm1(deg2, xp, W1, Pm):
    return pl.pallas_call(
        _mm1_body,
        grid=(NROWS // BM,),
        in_specs=[
            pl.BlockSpec((2, BM), lambda i: (0, i)),
            pl.BlockSpec((BM, F), lambda i: (i, 0)),
            pl.BlockSpec((F, F), lambda i: (0, 0)),
            pl.BlockSpec((FH, FH), lambda i: (0, 0)),
        ],
        out_specs=[
            pl.BlockSpec((BM, FH), lambda i: (i, 0)),
            pl.BlockSpec((BM, FH), lambda i: (i, 0)),
            pl.BlockSpec((BM, F), lambda i: (i, 0)),
        ],
        out_shape=[
            jax.ShapeDtypeStruct((NROWS, FH), jnp.bfloat16),
            jax.ShapeDtypeStruct((NROWS, FH), jnp.bfloat16),
            jax.ShapeDtypeStruct((NROWS, F), jnp.float32),
        ],
    )(deg2, xp, W1, Pm)


def _leaky(v):
    return jnp.where(v >= 0, v, NEG * v)


def _mm2_body(a_ref, z_ref, deg_ref, b_ref, w_ref, p_ref,
              h_ref, ya_ref, yb_ref, z2_ref):
    dinv = _dinv_of(deg_ref)
    a = a_ref[0] + a_ref[1]
    h = _leaky(a * dinv[:, None] + z_ref[...] + b_ref[...][None, :])
    h_ref[...] = h
    xw = jnp.dot(h, w_ref[...], preferred_element_type=jnp.float32)
    y = xw * dinv[:, None]
    pm = p_ref[...]
    ya_ref[...] = jnp.dot(y[:, :FH], pm,
                          preferred_element_type=jnp.float32).astype(jnp.bfloat16)
    yb_ref[...] = jnp.dot(y[:, FH:], pm,
                          preferred_element_type=jnp.float32).astype(jnp.bfloat16)
    z2_ref[...] = xw * (dinv * dinv)[:, None]


def _mm2(a1, z1, deg2, b1, W2, Pm):
    return pl.pallas_call(
        _mm2_body,
        grid=(NROWS // BM,),
        in_specs=[
            pl.BlockSpec((2, BM, F), lambda i: (0, i, 0)),
            pl.BlockSpec((BM, F), lambda i: (i, 0)),
            pl.BlockSpec((2, BM), lambda i: (0, i)),
            pl.BlockSpec((F,), lambda i: (0,)),
            pl.BlockSpec((F, F), lambda i: (0, 0)),
            pl.BlockSpec((FH, FH), lambda i: (0, 0)),
        ],
        out_specs=[
            pl.BlockSpec((BM, F), lambda i: (i, 0)),
            pl.BlockSpec((BM, FH), lambda i: (i, 0)),
            pl.BlockSpec((BM, FH), lambda i: (i, 0)),
            pl.BlockSpec((BM, F), lambda i: (i, 0)),
        ],
        out_shape=[
            jax.ShapeDtypeStruct((NROWS, F), jnp.float32),
            jax.ShapeDtypeStruct((NROWS, FH), jnp.bfloat16),
            jax.ShapeDtypeStruct((NROWS, FH), jnp.bfloat16),
            jax.ShapeDtypeStruct((NROWS, F), jnp.float32),
        ],
    )(a1, z1, deg2, b1, W2, Pm)


def _mm3_body(a_ref, z_ref, deg_ref, b_ref, h1_ref, wt_ref, wb_ref, bh_ref, o_ref):
    dinv = _dinv_of(deg_ref)
    a = a_ref[0] + a_ref[1]
    h2 = _leaky(a * dinv[:, None] + z_ref[...] + b_ref[...][None, :])
    o_ref[...] = (
        jnp.dot(h1_ref[...], wt_ref[...], preferred_element_type=jnp.float32)
        + jnp.dot(h2, wb_ref[...], preferred_element_type=jnp.float32)
        + bh_ref[...][None, :]
    )


def _mm3(a2, z2, deg2, b2, h1, Wt, Wb, bh):
    return pl.pallas_call(
        _mm3_body,
        grid=(NROWS // BM,),
        in_specs=[
            pl.BlockSpec((2, BM, F), lambda i: (0, i, 0)),
            pl.BlockSpec((BM, F), lambda i: (i, 0)),
            pl.BlockSpec((2, BM), lambda i: (0, i)),
            pl.BlockSpec((F,), lambda i: (0,)),
            pl.BlockSpec((BM, F), lambda i: (i, 0)),
            pl.BlockSpec((F, F), lambda i: (0, 0)),
            pl.BlockSpec((F, F), lambda i: (0, 0)),
            pl.BlockSpec((F,), lambda i: (0,)),
        ],
        out_specs=pl.BlockSpec((BM, F), lambda i: (i, 0)),
        out_shape=jax.ShapeDtypeStruct((N, F), jnp.float32),
    )(a2, z2, deg2, b2, h1, Wt, Wb, bh)


def kernel(x, edge_index, W1, b1, W2, b2, Wh, bh):
    row = edge_index[0]
    col = edge_index[1]
    pad = (TCH + CHMAX) * CB - E
    dummy = (jnp.arange(pad, dtype=jnp.int32) % (NROWS - N)) + N
    r3 = jnp.concatenate([row, jnp.zeros((pad,), jnp.int32)]).reshape(TCH + CHMAX, CB)
    c3 = jnp.concatenate([col, dummy]).reshape(TCH + CHMAX, CB)
    xp = jnp.pad(x, ((0, NROWS - N), (0, 0)))

    degp = _deg(c3)
    deg2 = degp.reshape(2, NROWS)
    Pm = jnp.asarray(_PMAT)

    y1a, y1b, z1 = _mm1(deg2, xp, W1, Pm)
    a1 = _agg(y1a, y1b, r3, c3)
    h1, y2a, y2b, z2 = _mm2(a1, z1, deg2, b1, W2, Pm)
    a2 = _agg(y2a, y2b, r3, c3)
    return _mm3(a2, z2, deg2, b2, h1, Wh[:F], Wh[F:], bh)
